# Initial kernel scaffold; baseline (speedup 1.0000x reference)
import jax
import jax.numpy as jnp
from jax.experimental import pallas as pl


def kernel(h, e, edge_index, Win_h, bin_h, Win_e, bin_e, A0, bA0, B0, bB0, C0, bC0, D0, bD0, Ew0, bEw0, A1, bA1, B1, bB1, C1, bC1, D1, bD1, Ew1, bEw1, Wmu, bmu, Wlv, blv):
    raise NotImplementedError("write your pallas kernel here")


# speedup vs baseline: 1.5314x; 1.5314x over previous
import functools

import jax
import jax.numpy as jnp
from jax import lax
from jax.experimental import pallas as pl
from jax.experimental.pallas import tpu as pltpu
from jax.experimental.pallas import tpu_sc as plsc

N = 50000
E = 800000
DIN_H = 32
DIN_E = 16
H = 64
Z = 32

NSUB = 16
E_PAD = 819200
NB_TOT = E_PAD // 128
NB_MAIN = NB_TOT // NSUB
CB = 8
NCH = NB_MAIN // CB
RB = 400
NRBLK = N // RB

BN = 2000
BE = 3200


def _dot(a, b):
    return jnp.dot(a, b, preferred_element_type=jnp.float32)


def _tc_node0_body(h_ref, win_ref, bin_ref, a_ref, ba_ref, b_ref, bb_ref,
                   d_ref, bd_ref, ew_ref, bew_ref, hh_o, ah_o, *pieces):
    hh = _dot(h_ref[...], win_ref[...]) + bin_ref[...]
    hh_o[...] = hh
    ah_o[...] = _dot(hh, a_ref[...]) + ba_ref[...]
    dh = _dot(hh, d_ref[...]) + bd_ref[...]
    eh = _dot(hh, ew_ref[...]) + bew_ref[...]
    bh = _dot(hh, b_ref[...]) + bb_ref[...]
    for g in range(4):
        pieces[g][...] = dh[:, g * 16:(g + 1) * 16]
        pieces[4 + g][...] = eh[:, g * 16:(g + 1) * 16]
        pieces[8 + g][...] = bh[:, g * 16:(g + 1) * 16]


def _tc_node1_body(hh_ref, ah_ref, n0, n1, n2, n3, dd0, dd1, dd2, dd3,
                   a_ref, ba_ref, b_ref, bb_ref, d_ref, bd_ref, ew_ref,
                   bew_ref, hh_o, ah_o, *pieces):
    num = jnp.concatenate([n0[...], n1[...], n2[...], n3[...]], axis=1)
    den = jnp.concatenate([dd0[...], dd1[...], dd2[...], dd3[...]], axis=1)
    hh = hh_ref[...] + jnp.maximum(ah_ref[...] + num / (den + 1e-6), 0.0)
    hh_o[...] = hh
    ah_o[...] = _dot(hh, a_ref[...]) + ba_ref[...]
    dh = _dot(hh, d_ref[...]) + bd_ref[...]
    eh = _dot(hh, ew_ref[...]) + bew_ref[...]
    bh = _dot(hh, b_ref[...]) + bb_ref[...]
    for g in range(4):
        pieces[g][...] = dh[:, g * 16:(g + 1) * 16]
        pieces[4 + g][...] = eh[:, g * 16:(g + 1) * 16]
        pieces[8 + g][...] = bh[:, g * 16:(g + 1) * 16]


def _tc_edge0_body(e_ref, w_ref, b_ref, *outs):
    ce = _dot(e_ref[...], w_ref[...]) + b_ref[...]
    ce = jnp.where(pl.program_id(0) < E // BE, ce, -1e30)
    for g in range(4):
        outs[g][...] = ce[:, g * 16:(g + 1) * 16]


def _tc_edge1_body(e_ref, r0, r1, r2, r3, w_ref, c_ref, b_ref, *outs):
    r = jnp.concatenate([r0[...], r1[...], r2[...], r3[...]], axis=1)
    ce = _dot(e_ref[...], w_ref[...]) + _dot(r, c_ref[...]) + b_ref[...]
    ce = jnp.where(pl.program_id(0) < E // BE, ce, -1e30)
    for g in range(4):
        outs[g][...] = ce[:, g * 16:(g + 1) * 16]


def _tc_final_body(hh_ref, ah_ref, n0, n1, n2, n3, dd0, dd1, dd2, dd3,
                   wmu_ref, bmu_ref, wlv_ref, blv_ref, eps_ref, z_o):
    num = jnp.concatenate([n0[...], n1[...], n2[...], n3[...]], axis=1)
    den = jnp.concatenate([dd0[...], dd1[...], dd2[...], dd3[...]], axis=1)
    hh = hh_ref[...] + jnp.maximum(ah_ref[...] + num / (den + 1e-6), 0.0)
    mu = _dot(hh, wmu_ref[...]) + bmu_ref[...]
    lv = _dot(hh, wlv_ref[...]) + blv_ref[...]
    std = jnp.exp(0.5 * lv) + 1e-6
    z_o[...] = mu + std * eps_ref[...]


def _nblk(r, c):
    return pl.BlockSpec((BN, c), lambda i: (i, 0))


def _wblk(r, c):
    return pl.BlockSpec((r, c), lambda i: (0, 0))


def _f32(shape):
    return jax.ShapeDtypeStruct(shape, jnp.float32)


def _tc_node0(h, Win_h, bin_h, A, bA, B, bB, D, bD, Ew, bEw):
    grid = (N // BN,)
    in_specs = [_nblk(BN, DIN_H), _wblk(DIN_H, H), _wblk(1, H)] + \
        [_wblk(H, H), _wblk(1, H)] * 4
    out_specs = [_nblk(BN, H), _nblk(BN, H)] + [_nblk(BN, 16)] * 12
    out_shape = [_f32((N, H)), _f32((N, H))] + [_f32((N, 16))] * 12
    return pl.pallas_call(
        _tc_node0_body, grid=grid, in_specs=in_specs,
        out_specs=out_specs, out_shape=out_shape)(
            h, Win_h, bin_h, A, bA, B, bB, D, bD, Ew, bEw)


def _tc_node1(hh, Ah, nps, dps, A, bA, B, bB, D, bD, Ew, bEw):
    grid = (N // BN,)
    in_specs = [_nblk(BN, H), _nblk(BN, H)] + [_nblk(BN, 16)] * 8 + \
        [_wblk(H, H), _wblk(1, H)] * 4
    out_specs = [_nblk(BN, H), _nblk(BN, H)] + [_nblk(BN, 16)] * 12
    out_shape = [_f32((N, H)), _f32((N, H))] + [_f32((N, 16))] * 12
    return pl.pallas_call(
        _tc_node1_body, grid=grid, in_specs=in_specs,
        out_specs=out_specs, out_shape=out_shape)(
            hh, Ah, *nps, *dps, A, bA, B, bB, D, bD, Ew, bEw)


def _eblk_in(c):
    return pl.BlockSpec((BE, c), lambda i: (jnp.minimum(i, E // BE - 1), 0))


def _tc_edge0(e, Wp, bp):
    grid = (E_PAD // BE,)
    in_specs = [_eblk_in(DIN_E), _wblk(DIN_E, H), _wblk(1, H)]
    out_specs = [pl.BlockSpec((BE, 16), lambda i: (i, 0))] * 4
    out_shape = [_f32((E_PAD, 16))] * 4
    return pl.pallas_call(
        _tc_edge0_body, grid=grid, in_specs=in_specs,
        out_specs=out_specs, out_shape=out_shape)(e, Wp, bp)


def _tc_edge1(e, rps, Wp, C, bp):
    grid = (E_PAD // BE,)
    in_specs = [_eblk_in(DIN_E)] + [_eblk_in(16)] * 4 + \
        [_wblk(DIN_E, H), _wblk(H, H), _wblk(1, H)]
    out_specs = [pl.BlockSpec((BE, 16), lambda i: (i, 0))] * 4
    out_shape = [_f32((E_PAD, 16))] * 4
    return pl.pallas_call(
        _tc_edge1_body, grid=grid, in_specs=in_specs,
        out_specs=out_specs, out_shape=out_shape)(e, *rps, Wp, C, bp)


def _tc_final(hh, Ah, nps, dps, Wmu, bmu, Wlv, blv, eps):
    grid = (N // BN,)
    in_specs = [_nblk(BN, H), _nblk(BN, H)] + [_nblk(BN, 16)] * 8 + \
        [_wblk(H, Z), _wblk(1, Z), _wblk(H, Z), _wblk(1, Z),
         _nblk(BN, Z)]
    out_specs = _nblk(BN, Z)
    return pl.pallas_call(
        _tc_final_body, grid=grid, in_specs=in_specs,
        out_specs=out_specs, out_shape=_f32((N, Z)))(
            hh, Ah, *nps, *dps, Wmu, bmu, Wlv, blv, eps)


def _zero_acc(s, zb, acc):
    z16 = jnp.zeros((16,), jnp.float32)

    def _zrow(i, carry):
        zb[i, :] = z16
        return carry

    lax.fori_loop(0, RB, _zrow, 0)

    def _zcopy(j, carry):
        blk = s + j * NSUB

        @pl.when(blk < NRBLK)
        def _():
            pltpu.sync_copy(zb, acc.at[pl.ds(blk * RB, RB)])

        return carry

    lax.fori_loop(0, (NRBLK + NSUB - 1) // NSUB, _zcopy, 0)


def _acc_readback(c, s, zb, acc, out_h):
    def _rb(j, carry):
        blk = s + j * NSUB

        @pl.when(blk < NRBLK)
        def _():
            base = blk * RB
            pltpu.sync_copy(acc.at[pl.ds(base, RB)], zb)
            pltpu.sync_copy(zb, out_h.at[pl.ds(c * N + base, RB)])

        return carry

    lax.fori_loop(0, (NRBLK + NSUB - 1) // NSUB, _rb, 0)


def _sc_a_body(write_R, src_h, dst_h, Da, Db, Ea, Eb, Ba, Bb, Cea, Ceb,
               *rest):
    if write_R:
        num_o, sga_o, sgb_o, Ra_o, Rb_o = rest[:5]
        scr = rest[5:]
    else:
        num_o, sga_o, sgb_o = rest[:3]
        Ra_o = Rb_o = None
        scr = rest[3:]
    sidx, didx, dbuf, ebuf, bbuf, cebuf, zb, accn, gsem, ssem = scr
    c = lax.axis_index("c")
    s = lax.axis_index("s")

    _zero_acc(s, zb, accn)
    plsc.subcore_barrier()

    def _do_batches(Dt, Et, Bt, Cet, sg_o, R_o, cb_base, nb):
        pltpu.sync_copy(src_h.at[pl.ds(cb_base, nb)], sidx.at[pl.ds(0, nb)])
        pltpu.sync_copy(dst_h.at[pl.ds(cb_base, nb)], didx.at[pl.ds(0, nb)])
        cps = []
        for jb in range(nb):
            cps.append(pltpu.async_copy(Dt.at[didx.at[jb]], dbuf.at[jb], gsem))
            cps.append(pltpu.async_copy(Et.at[sidx.at[jb]], ebuf.at[jb], gsem))
            cps.append(pltpu.async_copy(Bt.at[sidx.at[jb]], bbuf.at[jb], gsem))
            cps.append(pltpu.async_copy(
                Cet.at[pl.ds((cb_base + jb) * 128, 128)], cebuf.at[jb], gsem))
        for cp in cps:
            cp.wait()
        wr = []
        for jb in range(nb):
            def _row(i, carry, jb=jb):
                eh = dbuf[jb, i, :] + ebuf[jb, i, :] + cebuf[jb, i, :]
                sg = 1.0 / (1.0 + jnp.exp(-eh))
                b = bbuf[jb, i, :]
                bbuf[jb, i, :] = sg * b
                cebuf[jb, i, :] = sg
                if write_R:
                    dbuf[jb, i, :] = jnp.maximum(eh, 0.0)
                return carry

            lax.fori_loop(0, 128, _row, 0)
            pltpu.sync_copy(bbuf.at[jb], accn.at[didx.at[jb]], add=True)
            wr.append(pltpu.async_copy(
                cebuf.at[jb], sg_o.at[pl.ds((cb_base + jb) * 128, 128)],
                ssem))
            if write_R:
                wr.append(pltpu.async_copy(
                    dbuf.at[jb], R_o.at[pl.ds((cb_base + jb) * 128, 128)],
                    ssem))
        for cp in wr:
            cp.wait()

    def _do_edges(Dt, Et, Bt, Cet, sg_o, R_o):
        def _chunk(k, carry):
            _do_batches(Dt, Et, Bt, Cet, sg_o, R_o, s * NB_MAIN + k * CB, CB)
            return carry

        lax.fori_loop(0, NCH, _chunk, 0)

    @pl.when(c == 0)
    def _():
        _do_edges(Da, Ea, Ba, Cea, sga_o, Ra_o)

    @pl.when(c == 1)
    def _():
        _do_edges(Db, Eb, Bb, Ceb, sgb_o, Rb_o)

    plsc.subcore_barrier()
    _acc_readback(c, s, zb, accn, num_o)


def _sc_b_body(dst_h, sga, sgb, den_o, didx, cebuf, zb, accd, gsem):
    c = lax.axis_index("c")
    s = lax.axis_index("s")

    _zero_acc(s, zb, accd)
    plsc.subcore_barrier()

    def _do_batches(sgt, cb_base, nb):
        pltpu.sync_copy(dst_h.at[pl.ds(cb_base, nb)], didx.at[pl.ds(0, nb)])
        cps = []
        for jb in range(nb):
            cps.append(pltpu.async_copy(
                sgt.at[pl.ds((cb_base + jb) * 128, 128)], cebuf.at[jb], gsem))
        for cp in cps:
            cp.wait()
        for jb in range(nb):
            pltpu.sync_copy(cebuf.at[jb], accd.at[didx.at[jb]], add=True)

    def _do_edges(sgt):
        def _chunk(k, carry):
            _do_batches(sgt, s * NB_MAIN + k * CB, CB)
            return carry

        lax.fori_loop(0, NCH, _chunk, 0)

    @pl.when(c == 0)
    def _():
        _do_edges(sga)

    @pl.when(c == 1)
    def _():
        _do_edges(sgb)

    plsc.subcore_barrier()
    _acc_readback(c, s, zb, accd, den_o)


def _make_sc_a(write_R):
    mesh = plsc.VectorSubcoreMesh(core_axis_name="c", subcore_axis_name="s")
    out_type = [_f32((2 * N, 16)), _f32((E_PAD, 16)), _f32((E_PAD, 16))]
    if write_R:
        out_type += [_f32((E_PAD, 16)), _f32((E_PAD, 16))]
    scratch = [
        pltpu.VMEM((CB, 128), jnp.int32),
        pltpu.VMEM((CB, 128), jnp.int32),
        pltpu.VMEM((CB, 128, 16), jnp.float32),
        pltpu.VMEM((CB, 128, 16), jnp.float32),
        pltpu.VMEM((CB, 128, 16), jnp.float32),
        pltpu.VMEM((CB, 128, 16), jnp.float32),
        pltpu.VMEM((RB, 16), jnp.float32),
        pltpu.VMEM_SHARED((N, 16), jnp.float32),
        pltpu.SemaphoreType.DMA,
        pltpu.SemaphoreType.DMA,
    ]
    return pl.kernel(
        functools.partial(_sc_a_body, write_R),
        out_type=out_type, mesh=mesh, scratch_types=scratch,
        compiler_params=pltpu.CompilerParams(use_tc_tiling_on_sc=False))


def _make_sc_b():
    mesh = plsc.VectorSubcoreMesh(core_axis_name="c", subcore_axis_name="s")
    scratch = [
        pltpu.VMEM((CB, 128), jnp.int32),
        pltpu.VMEM((CB, 128, 16), jnp.float32),
        pltpu.VMEM((RB, 16), jnp.float32),
        pltpu.VMEM_SHARED((N, 16), jnp.float32),
        pltpu.SemaphoreType.DMA,
    ]
    return pl.kernel(
        _sc_b_body,
        out_type=[_f32((2 * N, 16))], mesh=mesh, scratch_types=scratch,
        compiler_params=pltpu.CompilerParams(use_tc_tiling_on_sc=False))


def kernel(h, e, edge_index, Win_h, bin_h, Win_e, bin_e,
           A0, bA0, B0, bB0, C0, bC0, D0, bD0, Ew0, bEw0,
           A1, bA1, B1, bB1, C1, bC1, D1, bD1, Ew1, bEw1,
           Wmu, bmu, Wlv, blv):
    pad = jnp.zeros((E_PAD - E,), jnp.int32)
    src2 = jnp.concatenate([edge_index[0], pad]).reshape(NB_TOT, 128)
    dst2 = jnp.concatenate([edge_index[1], pad]).reshape(NB_TOT, 128)

    W0p = Win_e @ C0
    b0p = (bin_e @ C0 + bC0).reshape(1, H)
    W1p = Win_e @ C1
    b1p = (bin_e @ C1 + bC1).reshape(1, H)
    bin_h2 = bin_h.reshape(1, H)
    eps = jax.random.normal(jax.random.key(42), (1, N, Z),
                            dtype=jnp.float32)

    outs = _tc_node0(h, Win_h, bin_h2,
                     A0, bA0.reshape(1, H), B0, bB0.reshape(1, H),
                     D0, bD0.reshape(1, H), Ew0, bEw0.reshape(1, H))
    hh, Ah0 = outs[0], outs[1]
    d0p, e0p, b0pieces = outs[2:6], outs[6:10], outs[10:14]

    ce0 = _tc_edge0(e, W0p, b0p)

    sc_a0 = _make_sc_a(True)
    sc_b = _make_sc_b()
    n01, sg0, sg1, R0, R1 = sc_a0(src2, dst2, d0p[0], d0p[1], e0p[0], e0p[1],
                                  b0pieces[0], b0pieces[1], ce0[0], ce0[1])
    n23, sg2, sg3, R2, R3 = sc_a0(src2, dst2, d0p[2], d0p[3], e0p[2], e0p[3],
                                  b0pieces[2], b0pieces[3], ce0[2], ce0[3])
    dd01 = sc_b(dst2, sg0, sg1)[0]
    dd23 = sc_b(dst2, sg2, sg3)[0]
    nps0 = (n01[:N], n01[N:], n23[:N], n23[N:])
    dps0 = (dd01[:N], dd01[N:], dd23[:N], dd23[N:])

    outs = _tc_node1(hh, Ah0, nps0, dps0,
                     A1, bA1.reshape(1, H), B1, bB1.reshape(1, H),
                     D1, bD1.reshape(1, H), Ew1, bEw1.reshape(1, H))
    hh1, Ah1 = outs[0], outs[1]
    d1p, e1p, b1pieces = outs[2:6], outs[6:10], outs[10:14]

    ce1 = _tc_edge1(e, (R0, R1, R2, R3), W1p, C1, b1p)

    sc_a1 = _make_sc_a(False)
    n01, sg0, sg1 = sc_a1(src2, dst2, d1p[0], d1p[1], e1p[0], e1p[1],
                          b1pieces[0], b1pieces[1], ce1[0], ce1[1])
    n23, sg2, sg3 = sc_a1(src2, dst2, d1p[2], d1p[3], e1p[2], e1p[3],
                          b1pieces[2], b1pieces[3], ce1[2], ce1[3])
    dd01 = sc_b(dst2, sg0, sg1)[0]
    dd23 = sc_b(dst2, sg2, sg3)[0]
    nps1 = (n01[:N], n01[N:], n23[:N], n23[N:])
    dps1 = (dd01[:N], dd01[N:], dd23[:N], dd23[N:])

    z = _tc_final(hh1, Ah1, nps1, dps1, Wmu, bmu.reshape(1, Z),
                  Wlv, blv.reshape(1, Z), eps[0])
    return z.reshape(1, N, Z)
